# Initial kernel scaffold; baseline (speedup 1.0000x reference)
import jax
import jax.numpy as jnp
from jax.experimental import pallas as pl


def kernel(x, edge_index, branch_W, branch_b, nl1_W, nl1_b, nl2_W, nl2_b, gates, temperature, out_W, out_b):
    raise NotImplementedError("write your pallas kernel here")


# speedup vs baseline: 5.3044x; 5.3044x over previous
import functools

import jax
import jax.numpy as jnp
from jax import lax
from jax.experimental import pallas as pl
from jax.experimental.pallas import tpu as pltpu
from jax.experimental.pallas import tpu_sc as plsc

N = 10000
E = 320000
D = 128
NB = 4
DAUG = 144
NPAD = 10240
NC, NS = 2, 16
NW = NC * NS
EPT = E // NW
C = 80
NCHUNK = EPT // C
RPT = NPAD // NS
BLK = 1024


def _sc_scatter_pass_body(table, row, col, zeros, out, idx_r, idx_c, rows_v,
                          acc, sem):
    cid = lax.axis_index("c")
    sid = lax.axis_index("s")
    wid = sid * NC + cid
    zbase = sid * RPT
    pltpu.sync_copy(zeros.at[pl.ds(zbase, RPT)], acc.at[pl.ds(zbase, RPT)])
    plsc.subcore_barrier()

    ebase = wid * EPT

    def body(j, carry):
        base = ebase + j * C
        pltpu.sync_copy(row.at[pl.ds(base, C)], idx_r)
        pltpu.sync_copy(col.at[pl.ds(base, C)], idx_c)
        pltpu.async_copy(table.at[idx_r], rows_v, sem).wait()
        pltpu.sync_copy(rows_v, acc.at[idx_c], add=True)
        return carry

    lax.fori_loop(0, NCHUNK, body, 0)
    plsc.subcore_barrier()
    obase = cid * NPAD + zbase
    pltpu.sync_copy(acc.at[pl.ds(zbase, RPT)], out.at[pl.ds(obase, RPT)])


@functools.cache
def _get_sc_scatter_pass():
    mesh = plsc.VectorSubcoreMesh(
        core_axis_name="c", subcore_axis_name="s",
        num_cores=NC, num_subcores=NS)
    return pl.kernel(
        _sc_scatter_pass_body,
        out_type=jax.ShapeDtypeStruct((NC * NPAD, DAUG), jnp.float32),
        mesh=mesh,
        scratch_types=[
            pltpu.VMEM((C,), jnp.int32),
            pltpu.VMEM((C,), jnp.int32),
            pltpu.VMEM((C, DAUG), jnp.float32),
            pltpu.VMEM_SHARED((NPAD, DAUG), jnp.float32),
            pltpu.SemaphoreType.DMA,
        ],
        compiler_params=pltpu.CompilerParams(use_tc_tiling_on_sc=False),
    )


def _combine1_body(s0, s1, z, inv):
    a = s0[0] + s1[0]
    cnt = a[:, D:D + 1]
    r = 1.0 / jnp.maximum(cnt, 1.0)
    z[...] = a * r
    inv[...] = r


_combine1 = pl.pallas_call(
    _combine1_body,
    grid=(NPAD // BLK,),
    in_specs=[
        pl.BlockSpec((1, BLK, DAUG), lambda b: (0, b, 0)),
        pl.BlockSpec((1, BLK, DAUG), lambda b: (1, b, 0)),
    ],
    out_specs=[
        pl.BlockSpec((BLK, DAUG), lambda b: (b, 0)),
        pl.BlockSpec((BLK, 1), lambda b: (b, 0)),
    ],
    out_shape=[
        jax.ShapeDtypeStruct((NPAD, DAUG), jnp.float32),
        jax.ShapeDtypeStruct((NPAD, 1), jnp.float32),
    ],
)


def _combine2_body(s0, s1, inv, z):
    a = s0[0] + s1[0]
    z[...] = a * inv[...]


_combine2 = pl.pallas_call(
    _combine2_body,
    grid=(NPAD // BLK,),
    in_specs=[
        pl.BlockSpec((1, BLK, DAUG), lambda b: (0, b, 0)),
        pl.BlockSpec((1, BLK, DAUG), lambda b: (1, b, 0)),
        pl.BlockSpec((BLK, 1), lambda b: (b, 0)),
    ],
    out_specs=pl.BlockSpec((BLK, DAUG), lambda b: (b, 0)),
    out_shape=jax.ShapeDtypeStruct((NPAD, DAUG), jnp.float32),
)


def _mmt(a, w):
    return lax.dot_general(a, w, (((1,), (1,)), ((), ())),
                           preferred_element_type=jnp.float32)


def _main_body(x, z1, z2, z3, bw, bb, n1w, n1b, n2w, n2b, owg, ob, o):
    acc = jnp.zeros((BLK, D), jnp.float32) + ob[...]
    zs = (None, z1, z2, z3)
    for i in range(NB):
        if i == 0:
            h = _mmt(x[...], bw[i]) + bb[i]
        else:
            zfull = zs[i][...]
            zi = zfull[:, :D]
            ui = zfull[:, D:D + 1]
            h = _mmt(zi, bw[i]) + ui * bb[i]
        h = _mmt(h, n1w[i]) + n1b[i]
        h = jnp.maximum(h, 0.0)
        h = _mmt(h, n2w[i]) + n2b[i]
        acc = acc + _mmt(h, owg[i])
    o[...] = acc


def _full(shape):
    nd = len(shape)
    return pl.BlockSpec(shape, lambda b, _n=nd: (0,) * _n)


_main = pl.pallas_call(
    _main_body,
    grid=(NPAD // BLK,),
    in_specs=[
        pl.BlockSpec((BLK, D), lambda b: (b, 0)),
        pl.BlockSpec((BLK, DAUG), lambda b: (b, 0)),
        pl.BlockSpec((BLK, DAUG), lambda b: (b, 0)),
        pl.BlockSpec((BLK, DAUG), lambda b: (b, 0)),
        _full((NB, D, D)),
        _full((NB, 1, D)),
        _full((NB, D, D)),
        _full((NB, 1, D)),
        _full((NB, D, D)),
        _full((NB, 1, D)),
        _full((NB, D, D)),
        _full((1, D)),
    ],
    out_specs=pl.BlockSpec((BLK, D), lambda b: (b, 0)),
    out_shape=jax.ShapeDtypeStruct((NPAD, D), jnp.float32),
)


def kernel(x, edge_index, branch_W, branch_b, nl1_W, nl1_b, nl2_W, nl2_b,
           gates, temperature, out_W, out_b):
    row = edge_index[0]
    col = edge_index[1]
    g = jax.nn.softmax(gates / temperature)

    xpad = jnp.zeros((NPAD, D), jnp.float32).at[:N].set(x)
    xaug = (jnp.zeros((NPAD, DAUG), jnp.float32)
            .at[:N, :D].set(x)
            .at[:N, D].set(1.0))
    zeros = jnp.zeros((NPAD, DAUG), jnp.float32)

    sc_pass = _get_sc_scatter_pass()
    s1 = sc_pass(xaug, row, col, zeros).reshape(NC, NPAD, DAUG)
    z1, inv = _combine1(s1, s1)
    s2 = sc_pass(z1, row, col, zeros).reshape(NC, NPAD, DAUG)
    z2 = _combine2(s2, s2, inv)
    s3 = sc_pass(z2, row, col, zeros).reshape(NC, NPAD, DAUG)
    z3 = _combine2(s3, s3, inv)

    owg = (out_W.reshape(D, NB, D).transpose(1, 0, 2)
           * g[:, None, None]).astype(jnp.float32)
    out = _main(xpad, z1, z2, z3,
                branch_W, branch_b.reshape(NB, 1, D),
                nl1_W, nl1_b.reshape(NB, 1, D),
                nl2_W, nl2_b.reshape(NB, 1, D),
                owg, out_b.reshape(1, D))
    return out[:N]


# speedup vs baseline: 9.0454x; 1.7053x over previous
import functools

import jax
import jax.numpy as jnp
from jax import lax
from jax.experimental import pallas as pl
from jax.experimental.pallas import tpu as pltpu
from jax.experimental.pallas import tpu_sc as plsc

N = 10000
E = 320000
D = 128
NB = 4
DAUG = 144
NPAD = 10000
NC, NS = 2, 16
NW = NC * NS
EPT = E // NW
C = 50
NCHUNK = EPT // C
RPT = NPAD // NS
BLK = 1000


def _sc_scatter_pass_body(table, row3, col3, zeros, out, idx_r, idx_c,
                          rows0, rows1, acc, semg0, semg1):
    cid = lax.axis_index("c")
    sid = lax.axis_index("s")
    wid = sid * NC + cid
    zbase = sid * RPT
    pltpu.sync_copy(zeros.at[pl.ds(zbase, RPT)], acc.at[pl.ds(zbase, RPT)])
    pltpu.sync_copy(row3.at[wid], idx_r)
    pltpu.sync_copy(col3.at[wid], idx_c)
    plsc.subcore_barrier()

    rows = (rows0, rows1)
    semg = (semg0, semg1)

    pltpu.async_copy(table.at[idx_r.at[0]], rows0, semg0)
    pltpu.async_copy(table.at[idx_r.at[1]], rows1, semg1)

    def body(p, carry):
        for b in range(2):
            c = 2 * p + b
            pltpu.make_async_copy(
                table.at[idx_r.at[c]], rows[b], semg[b]).wait()
            pltpu.sync_copy(rows[b], acc.at[idx_c.at[c]], add=True)

            @pl.when(c + 2 < NCHUNK)
            def _():
                pltpu.async_copy(table.at[idx_r.at[c + 2]], rows[b], semg[b])
        return carry

    lax.fori_loop(0, NCHUNK // 2, body, 0)
    plsc.subcore_barrier()
    obase = cid * NPAD + zbase
    pltpu.sync_copy(acc.at[pl.ds(zbase, RPT)], out.at[pl.ds(obase, RPT)])


@functools.cache
def _get_sc_scatter_pass():
    mesh = plsc.VectorSubcoreMesh(
        core_axis_name="c", subcore_axis_name="s",
        num_cores=NC, num_subcores=NS)
    return pl.kernel(
        _sc_scatter_pass_body,
        out_type=jax.ShapeDtypeStruct((NC * NPAD, DAUG), jnp.float32),
        mesh=mesh,
        scratch_types=[
            pltpu.VMEM((NCHUNK, C), jnp.int32),
            pltpu.VMEM((NCHUNK, C), jnp.int32),
            pltpu.VMEM((C, DAUG), jnp.float32),
            pltpu.VMEM((C, DAUG), jnp.float32),
            pltpu.VMEM_SHARED((NPAD, DAUG), jnp.float32),
            pltpu.SemaphoreType.DMA,
            pltpu.SemaphoreType.DMA,
        ],
        compiler_params=pltpu.CompilerParams(use_tc_tiling_on_sc=False),
    )


def _combine1_body(s0, s1, z, inv):
    a = s0[0] + s1[0]
    cnt = a[:, D:D + 1]
    r = 1.0 / jnp.maximum(cnt, 1.0)
    z[...] = a * r
    inv[...] = r


_combine1 = pl.pallas_call(
    _combine1_body,
    grid=(NPAD // BLK,),
    in_specs=[
        pl.BlockSpec((1, BLK, DAUG), lambda b: (0, b, 0)),
        pl.BlockSpec((1, BLK, DAUG), lambda b: (1, b, 0)),
    ],
    out_specs=[
        pl.BlockSpec((BLK, DAUG), lambda b: (b, 0)),
        pl.BlockSpec((BLK, 1), lambda b: (b, 0)),
    ],
    out_shape=[
        jax.ShapeDtypeStruct((NPAD, DAUG), jnp.float32),
        jax.ShapeDtypeStruct((NPAD, 1), jnp.float32),
    ],
)


def _combine2_body(s0, s1, inv, z):
    a = s0[0] + s1[0]
    z[...] = a * inv[...]


_combine2 = pl.pallas_call(
    _combine2_body,
    grid=(NPAD // BLK,),
    in_specs=[
        pl.BlockSpec((1, BLK, DAUG), lambda b: (0, b, 0)),
        pl.BlockSpec((1, BLK, DAUG), lambda b: (1, b, 0)),
        pl.BlockSpec((BLK, 1), lambda b: (b, 0)),
    ],
    out_specs=pl.BlockSpec((BLK, DAUG), lambda b: (b, 0)),
    out_shape=jax.ShapeDtypeStruct((NPAD, DAUG), jnp.float32),
)


def _mmt(a, w):
    return lax.dot_general(a, w, (((1,), (1,)), ((), ())),
                           preferred_element_type=jnp.float32)


def _main_body(x, z1, z2, z3, bw, bb, n1w, n1b, n2w, n2b, owg, ob, o):
    acc = jnp.zeros((BLK, D), jnp.float32) + ob[...]
    zs = (None, z1, z2, z3)
    for i in range(NB):
        if i == 0:
            h = _mmt(x[...], bw[i]) + bb[i]
        else:
            zfull = zs[i][...]
            zi = zfull[:, :D]
            ui = zfull[:, D:D + 1]
            h = _mmt(zi, bw[i]) + ui * bb[i]
        h = _mmt(h, n1w[i]) + n1b[i]
        h = jnp.maximum(h, 0.0)
        h = _mmt(h, n2w[i]) + n2b[i]
        acc = acc + _mmt(h, owg[i])
    o[...] = acc


def _full(shape):
    nd = len(shape)
    return pl.BlockSpec(shape, lambda b, _n=nd: (0,) * _n)


_main = pl.pallas_call(
    _main_body,
    grid=(NPAD // BLK,),
    in_specs=[
        pl.BlockSpec((BLK, D), lambda b: (b, 0)),
        pl.BlockSpec((BLK, DAUG), lambda b: (b, 0)),
        pl.BlockSpec((BLK, DAUG), lambda b: (b, 0)),
        pl.BlockSpec((BLK, DAUG), lambda b: (b, 0)),
        _full((NB, D, D)),
        _full((NB, 1, D)),
        _full((NB, D, D)),
        _full((NB, 1, D)),
        _full((NB, D, D)),
        _full((NB, 1, D)),
        _full((NB, D, D)),
        _full((1, D)),
    ],
    out_specs=pl.BlockSpec((BLK, D), lambda b: (b, 0)),
    out_shape=jax.ShapeDtypeStruct((NPAD, D), jnp.float32),
)


def kernel(x, edge_index, branch_W, branch_b, nl1_W, nl1_b, nl2_W, nl2_b,
           gates, temperature, out_W, out_b):
    row3 = edge_index[0].reshape(NW, NCHUNK, C)
    col3 = edge_index[1].reshape(NW, NCHUNK, C)
    g = jax.nn.softmax(gates / temperature)

    xaug = (jnp.zeros((NPAD, DAUG), jnp.float32)
            .at[:, :D].set(x)
            .at[:, D].set(1.0))
    zeros = jnp.zeros((NPAD, DAUG), jnp.float32)

    sc_pass = _get_sc_scatter_pass()
    s1 = sc_pass(xaug, row3, col3, zeros).reshape(NC, NPAD, DAUG)
    z1, inv = _combine1(s1, s1)
    s2 = sc_pass(z1, row3, col3, zeros).reshape(NC, NPAD, DAUG)
    z2 = _combine2(s2, s2, inv)
    s3 = sc_pass(z2, row3, col3, zeros).reshape(NC, NPAD, DAUG)
    z3 = _combine2(s3, s3, inv)

    owg = (out_W.reshape(D, NB, D).transpose(1, 0, 2)
           * g[:, None, None]).astype(jnp.float32)
    out = _main(x, z1, z2, z3,
                branch_W, branch_b.reshape(NB, 1, D),
                nl1_W, nl1_b.reshape(NB, 1, D),
                nl2_W, nl2_b.reshape(NB, 1, D),
                owg, out_b.reshape(1, D))
    return out


# speedup vs baseline: 11.4780x; 1.2689x over previous
import functools

import jax
import jax.numpy as jnp
from jax import lax
from jax.experimental import pallas as pl
from jax.experimental.pallas import tpu as pltpu
from jax.experimental.pallas import tpu_sc as plsc

N = 10000
E = 320000
D = 128
NB = 4
DAUG = 144
NPAD = 10000
NC, NS = 2, 16
NW = NC * NS
EPT = E // NW
C = 125
NCHUNK = EPT // C
RPT = NPAD // NS
BLK = 1000


def _sc_scatter_pass_body(table, idx4, out, idxb0, idxb1, idxb2, idxb3,
                          rows0, rows1, acc, semi0, semi1, semi2, semi3,
                          semg0, semg1):
    cid = lax.axis_index("c")
    sid = lax.axis_index("s")
    wid = sid * NC + cid
    zbase = sid * RPT

    def zbody(i, carry):
        for j in range(DAUG // 16):
            rows0[i, pl.ds(j * 16, 16)] = jnp.zeros((16,), jnp.float32)
        return carry

    lax.fori_loop(0, C, zbody, 0)
    for k in range(RPT // C):
        pltpu.sync_copy(rows0, acc.at[pl.ds(zbase + k * C, C)])
    plsc.subcore_barrier()

    idxs = (idxb0, idxb1, idxb2, idxb3)
    semi = (semi0, semi1, semi2, semi3)
    rows = (rows0, rows1)
    semg = (semg0, semg1)

    def idx_load(c, q):
        pltpu.async_copy(idx4.at[wid, c], idxs[q], semi[q])

    def idx_wait(c, q):
        pltpu.make_async_copy(idx4.at[wid, c], idxs[q], semi[q]).wait()

    def gather(q, b):
        pltpu.async_copy(table.at[idxs[q].at[0]], rows[b], semg[b])

    def gather_wait(q, b):
        pltpu.make_async_copy(table.at[idxs[q].at[0]], rows[b],
                              semg[b]).wait()

    for c in range(4):
        idx_load(c, c)
    for c in range(2):
        idx_wait(c, c)
        gather(c, c)

    def body(p, carry):
        for u in range(4):
            c = 4 * p + u
            b = u % 2
            gather_wait(u, b)
            pltpu.sync_copy(rows[b], acc.at[idxs[u].at[1]], add=True)

            @pl.when(c + 4 < NCHUNK)
            def _():
                idx_load(c + 4, u)

            @pl.when(c + 2 < NCHUNK)
            def _():
                idx_wait(c + 2, (u + 2) % 4)
                gather((u + 2) % 4, b)
        return carry

    lax.fori_loop(0, NCHUNK // 4, body, 0)
    plsc.subcore_barrier()
    obase = cid * NPAD + zbase
    pltpu.sync_copy(acc.at[pl.ds(zbase, RPT)], out.at[pl.ds(obase, RPT)])


@functools.cache
def _get_sc_scatter_pass():
    mesh = plsc.VectorSubcoreMesh(
        core_axis_name="c", subcore_axis_name="s",
        num_cores=NC, num_subcores=NS)
    return pl.kernel(
        _sc_scatter_pass_body,
        out_type=jax.ShapeDtypeStruct((NC * NPAD, DAUG), jnp.float32),
        mesh=mesh,
        scratch_types=[
            pltpu.VMEM((2, C), jnp.int32),
            pltpu.VMEM((2, C), jnp.int32),
            pltpu.VMEM((2, C), jnp.int32),
            pltpu.VMEM((2, C), jnp.int32),
            pltpu.VMEM((C, DAUG), jnp.float32),
            pltpu.VMEM((C, DAUG), jnp.float32),
            pltpu.VMEM_SHARED((NPAD, DAUG), jnp.float32),
            pltpu.SemaphoreType.DMA,
            pltpu.SemaphoreType.DMA,
            pltpu.SemaphoreType.DMA,
            pltpu.SemaphoreType.DMA,
            pltpu.SemaphoreType.DMA,
            pltpu.SemaphoreType.DMA,
        ],
        compiler_params=pltpu.CompilerParams(use_tc_tiling_on_sc=False),
    )


def _combine1_body(s0, s1, z, inv):
    a = s0[0] + s1[0]
    cnt = a[:, D:D + 1]
    r = 1.0 / jnp.maximum(cnt, 1.0)
    z[...] = a * r
    inv[...] = r


_combine1 = pl.pallas_call(
    _combine1_body,
    grid=(NPAD // BLK,),
    in_specs=[
        pl.BlockSpec((1, BLK, DAUG), lambda b: (0, b, 0)),
        pl.BlockSpec((1, BLK, DAUG), lambda b: (1, b, 0)),
    ],
    out_specs=[
        pl.BlockSpec((BLK, DAUG), lambda b: (b, 0)),
        pl.BlockSpec((BLK, 1), lambda b: (b, 0)),
    ],
    out_shape=[
        jax.ShapeDtypeStruct((NPAD, DAUG), jnp.float32),
        jax.ShapeDtypeStruct((NPAD, 1), jnp.float32),
    ],
)


def _combine2_body(s0, s1, inv, z):
    a = s0[0] + s1[0]
    z[...] = a * inv[...]


_combine2 = pl.pallas_call(
    _combine2_body,
    grid=(NPAD // BLK,),
    in_specs=[
        pl.BlockSpec((1, BLK, DAUG), lambda b: (0, b, 0)),
        pl.BlockSpec((1, BLK, DAUG), lambda b: (1, b, 0)),
        pl.BlockSpec((BLK, 1), lambda b: (b, 0)),
    ],
    out_specs=pl.BlockSpec((BLK, DAUG), lambda b: (b, 0)),
    out_shape=jax.ShapeDtypeStruct((NPAD, DAUG), jnp.float32),
)


def _mmt(a, w):
    return lax.dot_general(a, w, (((1,), (1,)), ((), ())),
                           preferred_element_type=jnp.float32)


def _main_body(x, z1, z2, z3, bw, bb, n1w, n1b, n2w, n2b, owg, ob, o):
    acc = jnp.zeros((BLK, D), jnp.float32) + ob[...]
    zs = (None, z1, z2, z3)
    for i in range(NB):
        if i == 0:
            h = _mmt(x[...], bw[i]) + bb[i]
        else:
            zfull = zs[i][...]
            zi = zfull[:, :D]
            ui = zfull[:, D:D + 1]
            h = _mmt(zi, bw[i]) + ui * bb[i]
        h = _mmt(h, n1w[i]) + n1b[i]
        h = jnp.maximum(h, 0.0)
        h = _mmt(h, n2w[i]) + n2b[i]
        acc = acc + _mmt(h, owg[i])
    o[...] = acc


def _full(shape):
    nd = len(shape)
    return pl.BlockSpec(shape, lambda b, _n=nd: (0,) * _n)


_main = pl.pallas_call(
    _main_body,
    grid=(NPAD // BLK,),
    in_specs=[
        pl.BlockSpec((BLK, D), lambda b: (b, 0)),
        pl.BlockSpec((BLK, DAUG), lambda b: (b, 0)),
        pl.BlockSpec((BLK, DAUG), lambda b: (b, 0)),
        pl.BlockSpec((BLK, DAUG), lambda b: (b, 0)),
        _full((NB, D, D)),
        _full((NB, 1, D)),
        _full((NB, D, D)),
        _full((NB, 1, D)),
        _full((NB, D, D)),
        _full((NB, 1, D)),
        _full((NB, D, D)),
        _full((1, D)),
    ],
    out_specs=pl.BlockSpec((BLK, D), lambda b: (b, 0)),
    out_shape=jax.ShapeDtypeStruct((NPAD, D), jnp.float32),
)


def kernel(x, edge_index, branch_W, branch_b, nl1_W, nl1_b, nl2_W, nl2_b,
           gates, temperature, out_W, out_b):
    idx4 = edge_index.reshape(2, NW, NCHUNK, C).transpose(1, 2, 0, 3)
    g = jax.nn.softmax(gates / temperature)

    xaug = (jnp.zeros((NPAD, DAUG), jnp.float32)
            .at[:, :D].set(x)
            .at[:, D].set(1.0))

    sc_pass = _get_sc_scatter_pass()
    s1 = sc_pass(xaug, idx4).reshape(NC, NPAD, DAUG)
    z1, inv = _combine1(s1, s1)
    s2 = sc_pass(z1, idx4).reshape(NC, NPAD, DAUG)
    z2 = _combine2(s2, s2, inv)
    s3 = sc_pass(z2, idx4).reshape(NC, NPAD, DAUG)
    z3 = _combine2(s3, s3, inv)

    owg = (out_W.reshape(D, NB, D).transpose(1, 0, 2)
           * g[:, None, None]).astype(jnp.float32)
    out = _main(x, z1, z2, z3,
                branch_W, branch_b.reshape(NB, 1, D),
                nl1_W, nl1_b.reshape(NB, 1, D),
                nl2_W, nl2_b.reshape(NB, 1, D),
                owg, out_b.reshape(1, D))
    return out


# speedup vs baseline: 11.7425x; 1.0230x over previous
import functools

import jax
import jax.numpy as jnp
from jax import lax
from jax.experimental import pallas as pl
from jax.experimental.pallas import tpu as pltpu
from jax.experimental.pallas import tpu_sc as plsc

N = 10000
E = 320000
D = 128
NB = 4
DAUG = 144
NPAD = 10000
NC, NS = 2, 16
NW = NC * NS
EPT = E // NW
C = 125
NCHUNK = EPT // C
RPT = NPAD // NS
BLK = 1000


def _sc_scatter_pass_body(table, idx4, out, idxb0, idxb1, idxb2, idxb3,
                          rows0, rows1, acc, semi0, semi1, semi2, semi3,
                          semg0, semg1):
    cid = lax.axis_index("c")
    sid = lax.axis_index("s")
    wid = sid * NC + cid
    zbase = sid * RPT

    def zbody(i, carry):
        for j in range(DAUG // 16):
            rows0[i, pl.ds(j * 16, 16)] = jnp.zeros((16,), jnp.float32)
        return carry

    lax.fori_loop(0, C, zbody, 0)
    for k in range(RPT // C):
        pltpu.sync_copy(rows0, acc.at[pl.ds(zbase + k * C, C)])
    plsc.subcore_barrier()

    idxs = (idxb0, idxb1, idxb2, idxb3)
    semi = (semi0, semi1, semi2, semi3)
    rows = (rows0, rows1)
    semg = (semg0, semg1)

    def idx_load(c, q):
        pltpu.async_copy(idx4.at[wid, c], idxs[q], semi[q])

    def idx_wait(c, q):
        pltpu.make_async_copy(idx4.at[wid, c], idxs[q], semi[q]).wait()

    def gather(q, b):
        pltpu.async_copy(table.at[idxs[q].at[0]], rows[b], semg[b])

    def gather_wait(q, b):
        pltpu.make_async_copy(table.at[idxs[q].at[0]], rows[b],
                              semg[b]).wait()

    for c in range(4):
        idx_load(c, c)
    for c in range(2):
        idx_wait(c, c)
        gather(c, c)

    def body(p, carry):
        for u in range(4):
            c = 4 * p + u
            b = u % 2
            gather_wait(u, b)
            pltpu.sync_copy(rows[b], acc.at[idxs[u].at[1]], add=True)

            @pl.when(c + 4 < NCHUNK)
            def _():
                idx_load(c + 4, u)

            @pl.when(c + 2 < NCHUNK)
            def _():
                idx_wait(c + 2, (u + 2) % 4)
                gather((u + 2) % 4, b)
        return carry

    lax.fori_loop(0, NCHUNK // 4, body, 0)
    plsc.subcore_barrier()
    obase = cid * NPAD + zbase
    pltpu.sync_copy(acc.at[pl.ds(zbase, RPT)], out.at[pl.ds(obase, RPT)])


@functools.cache
def _get_sc_scatter_pass():
    mesh = plsc.VectorSubcoreMesh(
        core_axis_name="c", subcore_axis_name="s",
        num_cores=NC, num_subcores=NS)
    return pl.kernel(
        _sc_scatter_pass_body,
        out_type=jax.ShapeDtypeStruct((NC * NPAD, DAUG), jnp.float32),
        mesh=mesh,
        scratch_types=[
            pltpu.VMEM((2, C), jnp.int32),
            pltpu.VMEM((2, C), jnp.int32),
            pltpu.VMEM((2, C), jnp.int32),
            pltpu.VMEM((2, C), jnp.int32),
            pltpu.VMEM((C, DAUG), jnp.float32),
            pltpu.VMEM((C, DAUG), jnp.float32),
            pltpu.VMEM_SHARED((NPAD, DAUG), jnp.float32),
            pltpu.SemaphoreType.DMA,
            pltpu.SemaphoreType.DMA,
            pltpu.SemaphoreType.DMA,
            pltpu.SemaphoreType.DMA,
            pltpu.SemaphoreType.DMA,
            pltpu.SemaphoreType.DMA,
        ],
        compiler_params=pltpu.CompilerParams(use_tc_tiling_on_sc=False),
    )


def _combine1_body(s0, s1, z, inv):
    a = s0[0] + s1[0]
    cnt = a[:, D:D + 1]
    r = 1.0 / jnp.maximum(cnt, 1.0)
    z[...] = a * r
    inv[...] = r


_combine1 = pl.pallas_call(
    _combine1_body,
    grid=(NPAD // BLK,),
    in_specs=[
        pl.BlockSpec((1, BLK, DAUG), lambda b: (0, b, 0)),
        pl.BlockSpec((1, BLK, DAUG), lambda b: (1, b, 0)),
    ],
    out_specs=[
        pl.BlockSpec((BLK, DAUG), lambda b: (b, 0)),
        pl.BlockSpec((BLK, 1), lambda b: (b, 0)),
    ],
    out_shape=[
        jax.ShapeDtypeStruct((NPAD, DAUG), jnp.float32),
        jax.ShapeDtypeStruct((NPAD, 1), jnp.float32),
    ],
)


def _combine2_body(s0, s1, inv, z):
    a = s0[0] + s1[0]
    z[...] = a * inv[...]


_combine2 = pl.pallas_call(
    _combine2_body,
    grid=(NPAD // BLK,),
    in_specs=[
        pl.BlockSpec((1, BLK, DAUG), lambda b: (0, b, 0)),
        pl.BlockSpec((1, BLK, DAUG), lambda b: (1, b, 0)),
        pl.BlockSpec((BLK, 1), lambda b: (b, 0)),
    ],
    out_specs=pl.BlockSpec((BLK, DAUG), lambda b: (b, 0)),
    out_shape=jax.ShapeDtypeStruct((NPAD, DAUG), jnp.float32),
)


def _mmt(a, w):
    return lax.dot_general(a, w, (((1,), (1,)), ((), ())),
                           preferred_element_type=jnp.float32)


def _branch(zi, ui, i, bw, bb, n1w, n1b, n2w, n2b, owg):
    if i == 0:
        h = _mmt(zi, bw[i]) + bb[i]
    else:
        h = _mmt(zi, bw[i]) + ui * bb[i]
    h = _mmt(h, n1w[i]) + n1b[i]
    h = jnp.maximum(h, 0.0)
    h = _mmt(h, n2w[i]) + n2b[i]
    return _mmt(h, owg[i])


def _main01_body(x, z1, bw, bb, n1w, n1b, n2w, n2b, owg, ob, o):
    acc = jnp.zeros((BLK, D), jnp.float32) + ob[...]
    z1f = z1[...]
    acc = acc + _branch(x[...], None, 0, bw, bb, n1w, n1b, n2w, n2b, owg)
    acc = acc + _branch(z1f[:, :D], z1f[:, D:D + 1], 1,
                        bw, bb, n1w, n1b, n2w, n2b, owg)
    o[...] = acc


def _main23_body(acc01, z2, s30, s31, inv, bw, bb, n1w, n1b, n2w, n2b, owg,
                 o):
    z3f = (s30[0] + s31[0]) * inv[...]
    z2f = z2[...]
    acc = acc01[...]
    acc = acc + _branch(z2f[:, :D], z2f[:, D:D + 1], 2,
                        bw, bb, n1w, n1b, n2w, n2b, owg)
    acc = acc + _branch(z3f[:, :D], z3f[:, D:D + 1], 3,
                        bw, bb, n1w, n1b, n2w, n2b, owg)
    o[...] = acc


def _full(shape):
    nd = len(shape)
    return pl.BlockSpec(shape, lambda b, _n=nd: (0,) * _n)


_W_SPECS = [
    _full((NB, D, D)),
    _full((NB, 1, D)),
    _full((NB, D, D)),
    _full((NB, 1, D)),
    _full((NB, D, D)),
    _full((NB, 1, D)),
    _full((NB, D, D)),
]

_main01 = pl.pallas_call(
    _main01_body,
    grid=(NPAD // BLK,),
    in_specs=[
        pl.BlockSpec((BLK, D), lambda b: (b, 0)),
        pl.BlockSpec((BLK, DAUG), lambda b: (b, 0)),
    ] + _W_SPECS + [_full((1, D))],
    out_specs=pl.BlockSpec((BLK, D), lambda b: (b, 0)),
    out_shape=jax.ShapeDtypeStruct((NPAD, D), jnp.float32),
)

_main23 = pl.pallas_call(
    _main23_body,
    grid=(NPAD // BLK,),
    in_specs=[
        pl.BlockSpec((BLK, D), lambda b: (b, 0)),
        pl.BlockSpec((BLK, DAUG), lambda b: (b, 0)),
        pl.BlockSpec((1, BLK, DAUG), lambda b: (0, b, 0)),
        pl.BlockSpec((1, BLK, DAUG), lambda b: (1, b, 0)),
        pl.BlockSpec((BLK, 1), lambda b: (b, 0)),
    ] + _W_SPECS,
    out_specs=pl.BlockSpec((BLK, D), lambda b: (b, 0)),
    out_shape=jax.ShapeDtypeStruct((NPAD, D), jnp.float32),
)


def kernel(x, edge_index, branch_W, branch_b, nl1_W, nl1_b, nl2_W, nl2_b,
           gates, temperature, out_W, out_b):
    idx4 = edge_index.reshape(2, NW, NCHUNK, C).transpose(1, 2, 0, 3)
    g = jax.nn.softmax(gates / temperature)

    xaug = (jnp.zeros((NPAD, DAUG), jnp.float32)
            .at[:, :D].set(x)
            .at[:, D].set(1.0))

    sc_pass = _get_sc_scatter_pass()
    s1 = sc_pass(xaug, idx4).reshape(NC, NPAD, DAUG)
    z1, inv = _combine1(s1, s1)
    s2 = sc_pass(z1, idx4).reshape(NC, NPAD, DAUG)
    z2 = _combine2(s2, s2, inv)
    s3 = sc_pass(z2, idx4).reshape(NC, NPAD, DAUG)

    owg = (out_W.reshape(D, NB, D).transpose(1, 0, 2)
           * g[:, None, None]).astype(jnp.float32)
    wargs = (branch_W, branch_b.reshape(NB, 1, D),
             nl1_W, nl1_b.reshape(NB, 1, D),
             nl2_W, nl2_b.reshape(NB, 1, D), owg)
    acc01 = _main01(x, z1, *wargs, out_b.reshape(1, D))
    out = _main23(acc01, z2, s3, s3, inv, *wargs)
    return out


# speedup vs baseline: 13.7541x; 1.1713x over previous
import functools

import jax
import jax.numpy as jnp
from jax import lax
from jax.experimental import pallas as pl
from jax.experimental.pallas import tpu as pltpu
from jax.experimental.pallas import tpu_sc as plsc

N = 10000
E = 320000
D = 128
NB = 4
DU = 16
NPAD = 10000
NC, NS = 2, 16
NW = NC * NS
EPT = E // NW
C = 125
NCHUNK = EPT // C
RPT = NPAD // NS
BLK = 1000


def _sc_scatter_pass_body(tf, tu, idx4, outf, outu, idxb0, idxb1, idxb2,
                          idxb3, rf0, rf1, ru0, ru1, accf, accu,
                          semi0, semi1, semi2, semi3, semg0, semg1):
    cid = lax.axis_index("c")
    sid = lax.axis_index("s")
    wid = sid * NC + cid
    zbase = sid * RPT

    def zf(i, carry):
        for j in range(D // 16):
            rf0[i, pl.ds(j * 16, 16)] = jnp.zeros((16,), jnp.float32)
        ru0[i, pl.ds(0, 16)] = jnp.zeros((16,), jnp.float32)
        return carry

    lax.fori_loop(0, C, zf, 0)
    for k in range(RPT // C):
        pltpu.sync_copy(rf0, accf.at[pl.ds(zbase + k * C, C)])
        pltpu.sync_copy(ru0, accu.at[pl.ds(zbase + k * C, C)])
    plsc.subcore_barrier()

    idxs = (idxb0, idxb1, idxb2, idxb3)
    semi = (semi0, semi1, semi2, semi3)
    rf = (rf0, rf1)
    ru = (ru0, ru1)
    semg = (semg0, semg1)

    def idx_load(c, q):
        pltpu.async_copy(idx4.at[wid, c], idxs[q], semi[q])

    def idx_wait(c, q):
        pltpu.make_async_copy(idx4.at[wid, c], idxs[q], semi[q]).wait()

    def gather(q, b):
        pltpu.async_copy(tf.at[idxs[q].at[0]], rf[b], semg[b])
        pltpu.async_copy(tu.at[idxs[q].at[0]], ru[b], semg[b])

    def gather_wait(q, b):
        pltpu.make_async_copy(tf.at[idxs[q].at[0]], rf[b], semg[b]).wait()
        pltpu.make_async_copy(tu.at[idxs[q].at[0]], ru[b], semg[b]).wait()

    for c in range(4):
        idx_load(c, c)
    for c in range(2):
        idx_wait(c, c)
        gather(c, c)

    def body(p, carry):
        for u in range(4):
            c = 4 * p + u
            b = u % 2
            gather_wait(u, b)
            pltpu.sync_copy(rf[b], accf.at[idxs[u].at[1]], add=True)
            pltpu.sync_copy(ru[b], accu.at[idxs[u].at[1]], add=True)

            @pl.when(c + 4 < NCHUNK)
            def _():
                idx_load(c + 4, u)

            @pl.when(c + 2 < NCHUNK)
            def _():
                idx_wait(c + 2, (u + 2) % 4)
                gather((u + 2) % 4, b)
        return carry

    lax.fori_loop(0, NCHUNK // 4, body, 0)
    plsc.subcore_barrier()
    obase = cid * NPAD + zbase
    pltpu.sync_copy(accf.at[pl.ds(zbase, RPT)], outf.at[pl.ds(obase, RPT)])
    pltpu.sync_copy(accu.at[pl.ds(zbase, RPT)], outu.at[pl.ds(obase, RPT)])


@functools.cache
def _get_sc_scatter_pass():
    mesh = plsc.VectorSubcoreMesh(
        core_axis_name="c", subcore_axis_name="s",
        num_cores=NC, num_subcores=NS)
    return pl.kernel(
        _sc_scatter_pass_body,
        out_type=(jax.ShapeDtypeStruct((NC * NPAD, D), jnp.float32),
                  jax.ShapeDtypeStruct((NC * NPAD, DU), jnp.float32)),
        mesh=mesh,
        scratch_types=[
            pltpu.VMEM((2, C), jnp.int32),
            pltpu.VMEM((2, C), jnp.int32),
            pltpu.VMEM((2, C), jnp.int32),
            pltpu.VMEM((2, C), jnp.int32),
            pltpu.VMEM((C, D), jnp.float32),
            pltpu.VMEM((C, D), jnp.float32),
            pltpu.VMEM((C, DU), jnp.float32),
            pltpu.VMEM((C, DU), jnp.float32),
            pltpu.VMEM_SHARED((NPAD, D), jnp.float32),
            pltpu.VMEM_SHARED((NPAD, DU), jnp.float32),
            pltpu.SemaphoreType.DMA,
            pltpu.SemaphoreType.DMA,
            pltpu.SemaphoreType.DMA,
            pltpu.SemaphoreType.DMA,
            pltpu.SemaphoreType.DMA,
            pltpu.SemaphoreType.DMA,
        ],
        compiler_params=pltpu.CompilerParams(use_tc_tiling_on_sc=False),
    )


def _combine1_body(sf0, sf1, su0, su1, tf, tu, inv):
    cnt = su0[0][:, 0:1] + su1[0][:, 0:1]
    r = 1.0 / jnp.maximum(cnt, 1.0)
    tf[...] = (sf0[0] + sf1[0]) * r
    tu[...] = (su0[0] + su1[0]) * r
    inv[...] = r


_combine1 = pl.pallas_call(
    _combine1_body,
    grid=(NPAD // BLK,),
    in_specs=[
        pl.BlockSpec((1, BLK, D), lambda b: (0, b, 0)),
        pl.BlockSpec((1, BLK, D), lambda b: (1, b, 0)),
        pl.BlockSpec((1, BLK, DU), lambda b: (0, b, 0)),
        pl.BlockSpec((1, BLK, DU), lambda b: (1, b, 0)),
    ],
    out_specs=[
        pl.BlockSpec((BLK, D), lambda b: (b, 0)),
        pl.BlockSpec((BLK, DU), lambda b: (b, 0)),
        pl.BlockSpec((BLK, 1), lambda b: (b, 0)),
    ],
    out_shape=[
        jax.ShapeDtypeStruct((NPAD, D), jnp.float32),
        jax.ShapeDtypeStruct((NPAD, DU), jnp.float32),
        jax.ShapeDtypeStruct((NPAD, 1), jnp.float32),
    ],
)


def _combine2_body(sf0, sf1, su0, su1, inv, tf, tu):
    r = inv[...]
    tf[...] = (sf0[0] + sf1[0]) * r
    tu[...] = (su0[0] + su1[0]) * r


_combine2 = pl.pallas_call(
    _combine2_body,
    grid=(NPAD // BLK,),
    in_specs=[
        pl.BlockSpec((1, BLK, D), lambda b: (0, b, 0)),
        pl.BlockSpec((1, BLK, D), lambda b: (1, b, 0)),
        pl.BlockSpec((1, BLK, DU), lambda b: (0, b, 0)),
        pl.BlockSpec((1, BLK, DU), lambda b: (1, b, 0)),
        pl.BlockSpec((BLK, 1), lambda b: (b, 0)),
    ],
    out_specs=[
        pl.BlockSpec((BLK, D), lambda b: (b, 0)),
        pl.BlockSpec((BLK, DU), lambda b: (b, 0)),
    ],
    out_shape=[
        jax.ShapeDtypeStruct((NPAD, D), jnp.float32),
        jax.ShapeDtypeStruct((NPAD, DU), jnp.float32),
    ],
)


def _mmt(a, w):
    return lax.dot_general(a, w, (((1,), (1,)), ((), ())),
                           preferred_element_type=jnp.float32)


def _branch(zi, ui, i, bw, bb, n1w, n1b, n2w, n2b, owg):
    if i == 0:
        h = _mmt(zi, bw[i]) + bb[i]
    else:
        h = _mmt(zi, bw[i]) + ui * bb[i]
    h = _mmt(h, n1w[i]) + n1b[i]
    h = jnp.maximum(h, 0.0)
    h = _mmt(h, n2w[i]) + n2b[i]
    return _mmt(h, owg[i])


def _main01_body(x, z1, u1, bw, bb, n1w, n1b, n2w, n2b, owg, ob, o):
    acc = jnp.zeros((BLK, D), jnp.float32) + ob[...]
    acc = acc + _branch(x[...], None, 0, bw, bb, n1w, n1b, n2w, n2b, owg)
    acc = acc + _branch(z1[...], u1[:, 0:1], 1,
                        bw, bb, n1w, n1b, n2w, n2b, owg)
    o[...] = acc


def _main23_body(acc01, z2, u2, sf30, sf31, su30, su31, inv,
                 bw, bb, n1w, n1b, n2w, n2b, owg, o):
    r = inv[...]
    z3 = (sf30[0] + sf31[0]) * r
    u3 = (su30[0][:, 0:1] + su31[0][:, 0:1]) * r
    acc = acc01[...]
    acc = acc + _branch(z2[...], u2[:, 0:1], 2,
                        bw, bb, n1w, n1b, n2w, n2b, owg)
    acc = acc + _branch(z3, u3, 3, bw, bb, n1w, n1b, n2w, n2b, owg)
    o[...] = acc


def _full(shape):
    nd = len(shape)
    return pl.BlockSpec(shape, lambda b, _n=nd: (0,) * _n)


_W_SPECS = [
    _full((NB, D, D)),
    _full((NB, 1, D)),
    _full((NB, D, D)),
    _full((NB, 1, D)),
    _full((NB, D, D)),
    _full((NB, 1, D)),
    _full((NB, D, D)),
]

_main01 = pl.pallas_call(
    _main01_body,
    grid=(NPAD // BLK,),
    in_specs=[
        pl.BlockSpec((BLK, D), lambda b: (b, 0)),
        pl.BlockSpec((BLK, D), lambda b: (b, 0)),
        pl.BlockSpec((BLK, DU), lambda b: (b, 0)),
    ] + _W_SPECS + [_full((1, D))],
    out_specs=pl.BlockSpec((BLK, D), lambda b: (b, 0)),
    out_shape=jax.ShapeDtypeStruct((NPAD, D), jnp.float32),
)

_main23 = pl.pallas_call(
    _main23_body,
    grid=(NPAD // BLK,),
    in_specs=[
        pl.BlockSpec((BLK, D), lambda b: (b, 0)),
        pl.BlockSpec((BLK, D), lambda b: (b, 0)),
        pl.BlockSpec((BLK, DU), lambda b: (b, 0)),
        pl.BlockSpec((1, BLK, D), lambda b: (0, b, 0)),
        pl.BlockSpec((1, BLK, D), lambda b: (1, b, 0)),
        pl.BlockSpec((1, BLK, DU), lambda b: (0, b, 0)),
        pl.BlockSpec((1, BLK, DU), lambda b: (1, b, 0)),
        pl.BlockSpec((BLK, 1), lambda b: (b, 0)),
    ] + _W_SPECS,
    out_specs=pl.BlockSpec((BLK, D), lambda b: (b, 0)),
    out_shape=jax.ShapeDtypeStruct((NPAD, D), jnp.float32),
)


def kernel(x, edge_index, branch_W, branch_b, nl1_W, nl1_b, nl2_W, nl2_b,
           gates, temperature, out_W, out_b):
    idx4 = edge_index.reshape(2, NW, NCHUNK, C).transpose(1, 2, 0, 3)
    g = jax.nn.softmax(gates / temperature)

    tu1 = jnp.zeros((NPAD, DU), jnp.float32).at[:, 0].set(1.0)

    sc_pass = _get_sc_scatter_pass()
    sf1, su1 = sc_pass(x, tu1, idx4)
    sf1 = sf1.reshape(NC, NPAD, D)
    su1 = su1.reshape(NC, NPAD, DU)
    z1, uu1, inv = _combine1(sf1, sf1, su1, su1)
    sf2, su2 = sc_pass(z1, uu1, idx4)
    sf2 = sf2.reshape(NC, NPAD, D)
    su2 = su2.reshape(NC, NPAD, DU)
    z2, uu2 = _combine2(sf2, sf2, su2, su2, inv)
    sf3, su3 = sc_pass(z2, uu2, idx4)
    sf3 = sf3.reshape(NC, NPAD, D)
    su3 = su3.reshape(NC, NPAD, DU)

    owg = (out_W.reshape(D, NB, D).transpose(1, 0, 2)
           * g[:, None, None]).astype(jnp.float32)
    wargs = (branch_W, branch_b.reshape(NB, 1, D),
             nl1_W, nl1_b.reshape(NB, 1, D),
             nl2_W, nl2_b.reshape(NB, 1, D), owg)
    acc01 = _main01(x, z1, uu1, *wargs, out_b.reshape(1, D))
    out = _main23(acc01, z2, uu2, sf3, sf3, su3, su3, inv, *wargs)
    return out


# speedup vs baseline: 13.7721x; 1.0013x over previous
import functools

import jax
import jax.numpy as jnp
from jax import lax
from jax.experimental import pallas as pl
from jax.experimental.pallas import tpu as pltpu
from jax.experimental.pallas import tpu_sc as plsc

N = 10000
E = 320000
D = 128
NB = 4
DU = 16
NPAD = 10000
NC, NS = 2, 16
NW = NC * NS
EPT = E // NW
C = 125
NCHUNK = EPT // C
RPT = NPAD // NS
BLK = 1000


def _sc_scatter_pass_body(tf, tu, idx4, outf, outu, idxb0, idxb1, idxb2,
                          idxb3, rf0, rf1, ru0, ru1, accf, accu,
                          semi0, semi1, semi2, semi3, semg0, semg1):
    cid = lax.axis_index("c")
    sid = lax.axis_index("s")
    wid = sid * NC + cid
    zbase = sid * RPT

    def zf(i, carry):
        for j in range(D // 16):
            rf0[i, pl.ds(j * 16, 16)] = jnp.zeros((16,), jnp.float32)
        ru0[i, pl.ds(0, 16)] = jnp.zeros((16,), jnp.float32)
        return carry

    lax.fori_loop(0, C, zf, 0)
    for k in range(RPT // C):
        pltpu.sync_copy(rf0, accf.at[pl.ds(zbase + k * C, C)])
        pltpu.sync_copy(ru0, accu.at[pl.ds(zbase + k * C, C)])
    plsc.subcore_barrier()

    idxs = (idxb0, idxb1, idxb2, idxb3)
    semi = (semi0, semi1, semi2, semi3)
    rf = (rf0, rf1)
    ru = (ru0, ru1)
    semg = (semg0, semg1)

    def idx_load(c, q):
        pltpu.async_copy(idx4.at[wid, c], idxs[q], semi[q])

    def idx_wait(c, q):
        pltpu.make_async_copy(idx4.at[wid, c], idxs[q], semi[q]).wait()

    def gather(q, b):
        pltpu.async_copy(tf.at[idxs[q].at[0]], rf[b], semg[b])
        pltpu.async_copy(tu.at[idxs[q].at[0]], ru[b], semg[b])

    def gather_wait(q, b):
        pltpu.make_async_copy(tf.at[idxs[q].at[0]], rf[b], semg[b]).wait()
        pltpu.make_async_copy(tu.at[idxs[q].at[0]], ru[b], semg[b]).wait()

    for c in range(4):
        idx_load(c, c)
    for c in range(2):
        idx_wait(c, c)
        gather(c, c)

    def body(p, carry):
        for u in range(4):
            c = 4 * p + u
            b = u % 2
            gather_wait(u, b)
            pltpu.sync_copy(rf[b], accf.at[idxs[u].at[1]], add=True)
            pltpu.sync_copy(ru[b], accu.at[idxs[u].at[1]], add=True)

            @pl.when(c + 4 < NCHUNK)
            def _():
                idx_load(c + 4, u)

            @pl.when(c + 2 < NCHUNK)
            def _():
                idx_wait(c + 2, (u + 2) % 4)
                gather((u + 2) % 4, b)
        return carry

    lax.fori_loop(0, NCHUNK // 4, body, 0)
    plsc.subcore_barrier()
    obase = cid * NPAD + zbase
    pltpu.sync_copy(accf.at[pl.ds(zbase, RPT)], outf.at[pl.ds(obase, RPT)])
    pltpu.sync_copy(accu.at[pl.ds(zbase, RPT)], outu.at[pl.ds(obase, RPT)])


@functools.cache
def _get_sc_scatter_pass():
    mesh = plsc.VectorSubcoreMesh(
        core_axis_name="c", subcore_axis_name="s",
        num_cores=NC, num_subcores=NS)
    return pl.kernel(
        _sc_scatter_pass_body,
        out_type=(jax.ShapeDtypeStruct((NC * NPAD, D), jnp.float32),
                  jax.ShapeDtypeStruct((NC * NPAD, DU), jnp.float32)),
        mesh=mesh,
        scratch_types=[
            pltpu.VMEM((2, C), jnp.int32),
            pltpu.VMEM((2, C), jnp.int32),
            pltpu.VMEM((2, C), jnp.int32),
            pltpu.VMEM((2, C), jnp.int32),
            pltpu.VMEM((C, D), jnp.float32),
            pltpu.VMEM((C, D), jnp.float32),
            pltpu.VMEM((C, DU), jnp.float32),
            pltpu.VMEM((C, DU), jnp.float32),
            pltpu.VMEM_SHARED((NPAD, D), jnp.float32),
            pltpu.VMEM_SHARED((NPAD, DU), jnp.float32),
            pltpu.SemaphoreType.DMA,
            pltpu.SemaphoreType.DMA,
            pltpu.SemaphoreType.DMA,
            pltpu.SemaphoreType.DMA,
            pltpu.SemaphoreType.DMA,
            pltpu.SemaphoreType.DMA,
        ],
        compiler_params=pltpu.CompilerParams(use_tc_tiling_on_sc=False),
    )


def _combine1_body(sf0, sf1, su0, su1, tf, tu, inv):
    cnt = su0[:, 0:1] + su1[:, 0:1]
    r = 1.0 / jnp.maximum(cnt, 1.0)
    tf[...] = (sf0[...] + sf1[...]) * r
    tu[...] = (su0[...] + su1[...]) * r
    inv[...] = r


_combine1 = pl.pallas_call(
    _combine1_body,
    grid=(NPAD // BLK,),
    in_specs=[
        pl.BlockSpec((BLK, D), lambda b: (b, 0)),
        pl.BlockSpec((BLK, D), lambda b: (NPAD // BLK + b, 0)),
        pl.BlockSpec((BLK, DU), lambda b: (b, 0)),
        pl.BlockSpec((BLK, DU), lambda b: (NPAD // BLK + b, 0)),
    ],
    out_specs=[
        pl.BlockSpec((BLK, D), lambda b: (b, 0)),
        pl.BlockSpec((BLK, DU), lambda b: (b, 0)),
        pl.BlockSpec((BLK, 1), lambda b: (b, 0)),
    ],
    out_shape=[
        jax.ShapeDtypeStruct((NPAD, D), jnp.float32),
        jax.ShapeDtypeStruct((NPAD, DU), jnp.float32),
        jax.ShapeDtypeStruct((NPAD, 1), jnp.float32),
    ],
)


def _combine2_body(sf0, sf1, su0, su1, inv, tf, tu):
    r = inv[...]
    tf[...] = (sf0[...] + sf1[...]) * r
    tu[...] = (su0[...] + su1[...]) * r


_combine2 = pl.pallas_call(
    _combine2_body,
    grid=(NPAD // BLK,),
    in_specs=[
        pl.BlockSpec((BLK, D), lambda b: (b, 0)),
        pl.BlockSpec((BLK, D), lambda b: (NPAD // BLK + b, 0)),
        pl.BlockSpec((BLK, DU), lambda b: (b, 0)),
        pl.BlockSpec((BLK, DU), lambda b: (NPAD // BLK + b, 0)),
        pl.BlockSpec((BLK, 1), lambda b: (b, 0)),
    ],
    out_specs=[
        pl.BlockSpec((BLK, D), lambda b: (b, 0)),
        pl.BlockSpec((BLK, DU), lambda b: (b, 0)),
    ],
    out_shape=[
        jax.ShapeDtypeStruct((NPAD, D), jnp.float32),
        jax.ShapeDtypeStruct((NPAD, DU), jnp.float32),
    ],
)


def _mmt(a, w):
    return lax.dot_general(a, w, (((1,), (1,)), ((), ())),
                           preferred_element_type=jnp.float32)


def _branch(zi, ui, i, bw, bb, n1w, n1b, n2w, n2b, owg):
    if i == 0:
        h = _mmt(zi, bw[i]) + bb[i]
    else:
        h = _mmt(zi, bw[i]) + ui * bb[i]
    h = _mmt(h, n1w[i]) + n1b[i]
    h = jnp.maximum(h, 0.0)
    h = _mmt(h, n2w[i]) + n2b[i]
    return _mmt(h, owg[i])


def _main01_body(x, z1, u1, bw, bb, n1w, n1b, n2w, n2b, owg, ob, o):
    acc = jnp.zeros((BLK, D), jnp.float32) + ob[...]
    acc = acc + _branch(x[...], None, 0, bw, bb, n1w, n1b, n2w, n2b, owg)
    acc = acc + _branch(z1[...], u1[:, 0:1], 1,
                        bw, bb, n1w, n1b, n2w, n2b, owg)
    o[...] = acc


def _main23_body(acc01, z2, u2, sf30, sf31, su30, su31, inv,
                 bw, bb, n1w, n1b, n2w, n2b, owg, o):
    r = inv[...]
    z3 = (sf30[...] + sf31[...]) * r
    u3 = (su30[:, 0:1] + su31[:, 0:1]) * r
    acc = acc01[...]
    acc = acc + _branch(z2[...], u2[:, 0:1], 2,
                        bw, bb, n1w, n1b, n2w, n2b, owg)
    acc = acc + _branch(z3, u3, 3, bw, bb, n1w, n1b, n2w, n2b, owg)
    o[...] = acc


def _full(shape):
    nd = len(shape)
    return pl.BlockSpec(shape, lambda b, _n=nd: (0,) * _n)


_W_SPECS = [
    _full((NB, D, D)),
    _full((NB, 1, D)),
    _full((NB, D, D)),
    _full((NB, 1, D)),
    _full((NB, D, D)),
    _full((NB, 1, D)),
    _full((NB, D, D)),
]

_main01 = pl.pallas_call(
    _main01_body,
    grid=(NPAD // BLK,),
    in_specs=[
        pl.BlockSpec((BLK, D), lambda b: (b, 0)),
        pl.BlockSpec((BLK, D), lambda b: (b, 0)),
        pl.BlockSpec((BLK, DU), lambda b: (b, 0)),
    ] + _W_SPECS + [_full((1, D))],
    out_specs=pl.BlockSpec((BLK, D), lambda b: (b, 0)),
    out_shape=jax.ShapeDtypeStruct((NPAD, D), jnp.float32),
)

_main23 = pl.pallas_call(
    _main23_body,
    grid=(NPAD // BLK,),
    in_specs=[
        pl.BlockSpec((BLK, D), lambda b: (b, 0)),
        pl.BlockSpec((BLK, D), lambda b: (b, 0)),
        pl.BlockSpec((BLK, DU), lambda b: (b, 0)),
        pl.BlockSpec((BLK, D), lambda b: (b, 0)),
        pl.BlockSpec((BLK, D), lambda b: (NPAD // BLK + b, 0)),
        pl.BlockSpec((BLK, DU), lambda b: (b, 0)),
        pl.BlockSpec((BLK, DU), lambda b: (NPAD // BLK + b, 0)),
        pl.BlockSpec((BLK, 1), lambda b: (b, 0)),
    ] + _W_SPECS,
    out_specs=pl.BlockSpec((BLK, D), lambda b: (b, 0)),
    out_shape=jax.ShapeDtypeStruct((NPAD, D), jnp.float32),
)


def kernel(x, edge_index, branch_W, branch_b, nl1_W, nl1_b, nl2_W, nl2_b,
           gates, temperature, out_W, out_b):
    idx4 = edge_index.reshape(2, NW, NCHUNK, C).transpose(1, 2, 0, 3)
    g = jax.nn.softmax(gates / temperature)

    tu1 = jnp.zeros((NPAD, DU), jnp.float32).at[:, 0].set(1.0)

    sc_pass = _get_sc_scatter_pass()
    sf1, su1 = sc_pass(x, tu1, idx4)
    z1, uu1, inv = _combine1(sf1, sf1, su1, su1)
    sf2, su2 = sc_pass(z1, uu1, idx4)
    z2, uu2 = _combine2(sf2, sf2, su2, su2, inv)
    sf3, su3 = sc_pass(z2, uu2, idx4)

    owg = (out_W.reshape(D, NB, D).transpose(1, 0, 2)
           * g[:, None, None]).astype(jnp.float32)
    wargs = (branch_W, branch_b.reshape(NB, 1, D),
             nl1_W, nl1_b.reshape(NB, 1, D),
             nl2_W, nl2_b.reshape(NB, 1, D), owg)
    acc01 = _main01(x, z1, uu1, *wargs, out_b.reshape(1, D))
    out = _main23(acc01, z2, uu2, sf3, sf3, su3, su3, inv, *wargs)
    return out


# speedup vs baseline: 13.9876x; 1.0157x over previous
import functools

import jax
import jax.numpy as jnp
from jax import lax
from jax.experimental import pallas as pl
from jax.experimental.pallas import tpu as pltpu
from jax.experimental.pallas import tpu_sc as plsc

N = 10000
E = 320000
D = 128
NB = 4
DU = 16
NPAD = 10000
NC, NS = 2, 16
NW = NC * NS
EPT = E // NW
C = 125
NCHUNK = EPT // C
RPT = NPAD // NS
BLK = 2000


def _sc_scatter_pass_body(tf, tu, idx4, outf, outu, idxb0, idxb1, idxb2,
                          idxb3, rf0, rf1, ru0, ru1, accf, accu,
                          semi0, semi1, semi2, semi3, semg0, semg1):
    cid = lax.axis_index("c")
    sid = lax.axis_index("s")
    wid = sid * NC + cid
    zbase = sid * RPT

    def zf(i, carry):
        for j in range(D // 16):
            rf0[i, pl.ds(j * 16, 16)] = jnp.zeros((16,), jnp.float32)
        ru0[i, pl.ds(0, 16)] = jnp.zeros((16,), jnp.float32)
        return carry

    lax.fori_loop(0, C, zf, 0)
    for k in range(RPT // C):
        pltpu.sync_copy(rf0, accf.at[pl.ds(zbase + k * C, C)])
        pltpu.sync_copy(ru0, accu.at[pl.ds(zbase + k * C, C)])
    plsc.subcore_barrier()

    idxs = (idxb0, idxb1, idxb2, idxb3)
    semi = (semi0, semi1, semi2, semi3)
    rf = (rf0, rf1)
    ru = (ru0, ru1)
    semg = (semg0, semg1)

    def idx_load(c, q):
        pltpu.async_copy(idx4.at[wid, c], idxs[q], semi[q])

    def idx_wait(c, q):
        pltpu.make_async_copy(idx4.at[wid, c], idxs[q], semi[q]).wait()

    def gather(q, b):
        pltpu.async_copy(tf.at[idxs[q].at[0]], rf[b], semg[b])
        pltpu.async_copy(tu.at[idxs[q].at[0]], ru[b], semg[b])

    def gather_wait(q, b):
        pltpu.make_async_copy(tf.at[idxs[q].at[0]], rf[b], semg[b]).wait()
        pltpu.make_async_copy(tu.at[idxs[q].at[0]], ru[b], semg[b]).wait()

    for c in range(4):
        idx_load(c, c)
    for c in range(2):
        idx_wait(c, c)
        gather(c, c)

    def body(p, carry):
        for u in range(4):
            c = 4 * p + u
            b = u % 2
            gather_wait(u, b)
            pltpu.sync_copy(rf[b], accf.at[idxs[u].at[1]], add=True)
            pltpu.sync_copy(ru[b], accu.at[idxs[u].at[1]], add=True)

            @pl.when(c + 4 < NCHUNK)
            def _():
                idx_load(c + 4, u)

            @pl.when(c + 2 < NCHUNK)
            def _():
                idx_wait(c + 2, (u + 2) % 4)
                gather((u + 2) % 4, b)
        return carry

    lax.fori_loop(0, NCHUNK // 4, body, 0)
    plsc.subcore_barrier()
    obase = cid * NPAD + zbase
    pltpu.sync_copy(accf.at[pl.ds(zbase, RPT)], outf.at[pl.ds(obase, RPT)])
    pltpu.sync_copy(accu.at[pl.ds(zbase, RPT)], outu.at[pl.ds(obase, RPT)])


@functools.cache
def _get_sc_scatter_pass():
    mesh = plsc.VectorSubcoreMesh(
        core_axis_name="c", subcore_axis_name="s",
        num_cores=NC, num_subcores=NS)
    return pl.kernel(
        _sc_scatter_pass_body,
        out_type=(jax.ShapeDtypeStruct((NC * NPAD, D), jnp.float32),
                  jax.ShapeDtypeStruct((NC * NPAD, DU), jnp.float32)),
        mesh=mesh,
        scratch_types=[
            pltpu.VMEM((2, C), jnp.int32),
            pltpu.VMEM((2, C), jnp.int32),
            pltpu.VMEM((2, C), jnp.int32),
            pltpu.VMEM((2, C), jnp.int32),
            pltpu.VMEM((C, D), jnp.float32),
            pltpu.VMEM((C, D), jnp.float32),
            pltpu.VMEM((C, DU), jnp.float32),
            pltpu.VMEM((C, DU), jnp.float32),
            pltpu.VMEM_SHARED((NPAD, D), jnp.float32),
            pltpu.VMEM_SHARED((NPAD, DU), jnp.float32),
            pltpu.SemaphoreType.DMA,
            pltpu.SemaphoreType.DMA,
            pltpu.SemaphoreType.DMA,
            pltpu.SemaphoreType.DMA,
            pltpu.SemaphoreType.DMA,
            pltpu.SemaphoreType.DMA,
        ],
        compiler_params=pltpu.CompilerParams(use_tc_tiling_on_sc=False),
    )


def _combine1_body(sf0, sf1, su0, su1, tf, tu, inv):
    cnt = su0[:, 0:1] + su1[:, 0:1]
    r = 1.0 / jnp.maximum(cnt, 1.0)
    tf[...] = (sf0[...] + sf1[...]) * r
    tu[...] = (su0[...] + su1[...]) * r
    inv[...] = r


_combine1 = pl.pallas_call(
    _combine1_body,
    grid=(NPAD // BLK,),
    in_specs=[
        pl.BlockSpec((BLK, D), lambda b: (b, 0)),
        pl.BlockSpec((BLK, D), lambda b: (NPAD // BLK + b, 0)),
        pl.BlockSpec((BLK, DU), lambda b: (b, 0)),
        pl.BlockSpec((BLK, DU), lambda b: (NPAD // BLK + b, 0)),
    ],
    out_specs=[
        pl.BlockSpec((BLK, D), lambda b: (b, 0)),
        pl.BlockSpec((BLK, DU), lambda b: (b, 0)),
        pl.BlockSpec((BLK, 1), lambda b: (b, 0)),
    ],
    out_shape=[
        jax.ShapeDtypeStruct((NPAD, D), jnp.float32),
        jax.ShapeDtypeStruct((NPAD, DU), jnp.float32),
        jax.ShapeDtypeStruct((NPAD, 1), jnp.float32),
    ],
)


def _combine2_body(sf0, sf1, su0, su1, inv, tf, tu):
    r = inv[...]
    tf[...] = (sf0[...] + sf1[...]) * r
    tu[...] = (su0[...] + su1[...]) * r


_combine2 = pl.pallas_call(
    _combine2_body,
    grid=(NPAD // BLK,),
    in_specs=[
        pl.BlockSpec((BLK, D), lambda b: (b, 0)),
        pl.BlockSpec((BLK, D), lambda b: (NPAD // BLK + b, 0)),
        pl.BlockSpec((BLK, DU), lambda b: (b, 0)),
        pl.BlockSpec((BLK, DU), lambda b: (NPAD // BLK + b, 0)),
        pl.BlockSpec((BLK, 1), lambda b: (b, 0)),
    ],
    out_specs=[
        pl.BlockSpec((BLK, D), lambda b: (b, 0)),
        pl.BlockSpec((BLK, DU), lambda b: (b, 0)),
    ],
    out_shape=[
        jax.ShapeDtypeStruct((NPAD, D), jnp.float32),
        jax.ShapeDtypeStruct((NPAD, DU), jnp.float32),
    ],
)


def _mmt(a, w):
    return lax.dot_general(a, w, (((1,), (1,)), ((), ())),
                           preferred_element_type=jnp.float32)


def _branch(zi, ui, i, bw, bb, n1w, n1b, n2w, n2b, owg):
    if i == 0:
        h = _mmt(zi, bw[i]) + bb[i]
    else:
        h = _mmt(zi, bw[i]) + ui * bb[i]
    h = _mmt(h, n1w[i]) + n1b[i]
    h = jnp.maximum(h, 0.0)
    h = _mmt(h, n2w[i]) + n2b[i]
    return _mmt(h, owg[i])


def _main01_body(x, z1, u1, bw, bb, n1w, n1b, n2w, n2b, owg, ob, o):
    acc = jnp.zeros((BLK, D), jnp.float32) + ob[...]
    acc = acc + _branch(x[...], None, 0, bw, bb, n1w, n1b, n2w, n2b, owg)
    acc = acc + _branch(z1[...], u1[:, 0:1], 1,
                        bw, bb, n1w, n1b, n2w, n2b, owg)
    o[...] = acc


def _main2_body(acc01, z2, u2, bw, bb, n1w, n1b, n2w, n2b, owg, o):
    o[...] = acc01[...] + _branch(z2[...], u2[:, 0:1], 2,
                                  bw, bb, n1w, n1b, n2w, n2b, owg)


def _main3_body(acc012, sf30, sf31, su30, su31, inv,
                bw, bb, n1w, n1b, n2w, n2b, owg, o):
    r = inv[...]
    z3 = (sf30[...] + sf31[...]) * r
    u3 = (su30[:, 0:1] + su31[:, 0:1]) * r
    o[...] = acc012[...] + _branch(z3, u3, 3,
                                   bw, bb, n1w, n1b, n2w, n2b, owg)


def _full(shape):
    nd = len(shape)
    return pl.BlockSpec(shape, lambda b, _n=nd: (0,) * _n)


_W_SPECS = [
    _full((NB, D, D)),
    _full((NB, 1, D)),
    _full((NB, D, D)),
    _full((NB, 1, D)),
    _full((NB, D, D)),
    _full((NB, 1, D)),
    _full((NB, D, D)),
]

_main01 = pl.pallas_call(
    _main01_body,
    grid=(NPAD // BLK,),
    in_specs=[
        pl.BlockSpec((BLK, D), lambda b: (b, 0)),
        pl.BlockSpec((BLK, D), lambda b: (b, 0)),
        pl.BlockSpec((BLK, DU), lambda b: (b, 0)),
    ] + _W_SPECS + [_full((1, D))],
    out_specs=pl.BlockSpec((BLK, D), lambda b: (b, 0)),
    out_shape=jax.ShapeDtypeStruct((NPAD, D), jnp.float32),
)

_main2 = pl.pallas_call(
    _main2_body,
    grid=(NPAD // BLK,),
    in_specs=[
        pl.BlockSpec((BLK, D), lambda b: (b, 0)),
        pl.BlockSpec((BLK, D), lambda b: (b, 0)),
        pl.BlockSpec((BLK, DU), lambda b: (b, 0)),
    ] + _W_SPECS,
    out_specs=pl.BlockSpec((BLK, D), lambda b: (b, 0)),
    out_shape=jax.ShapeDtypeStruct((NPAD, D), jnp.float32),
)

_main3 = pl.pallas_call(
    _main3_body,
    grid=(NPAD // BLK,),
    in_specs=[
        pl.BlockSpec((BLK, D), lambda b: (b, 0)),
        pl.BlockSpec((BLK, D), lambda b: (b, 0)),
        pl.BlockSpec((BLK, D), lambda b: (NPAD // BLK + b, 0)),
        pl.BlockSpec((BLK, DU), lambda b: (b, 0)),
        pl.BlockSpec((BLK, DU), lambda b: (NPAD // BLK + b, 0)),
        pl.BlockSpec((BLK, 1), lambda b: (b, 0)),
    ] + _W_SPECS,
    out_specs=pl.BlockSpec((BLK, D), lambda b: (b, 0)),
    out_shape=jax.ShapeDtypeStruct((NPAD, D), jnp.float32),
)


def kernel(x, edge_index, branch_W, branch_b, nl1_W, nl1_b, nl2_W, nl2_b,
           gates, temperature, out_W, out_b):
    idx4 = edge_index.reshape(2, NW, NCHUNK, C).transpose(1, 2, 0, 3)
    g = jax.nn.softmax(gates / temperature)

    tu1 = jnp.zeros((NPAD, DU), jnp.float32).at[:, 0].set(1.0)

    sc_pass = _get_sc_scatter_pass()
    sf1, su1 = sc_pass(x, tu1, idx4)
    z1, uu1, inv = _combine1(sf1, sf1, su1, su1)
    sf2, su2 = sc_pass(z1, uu1, idx4)
    z2, uu2 = _combine2(sf2, sf2, su2, su2, inv)
    sf3, su3 = sc_pass(z2, uu2, idx4)

    owg = (out_W.reshape(D, NB, D).transpose(1, 0, 2)
           * g[:, None, None]).astype(jnp.float32)
    wargs = (branch_W, branch_b.reshape(NB, 1, D),
             nl1_W, nl1_b.reshape(NB, 1, D),
             nl2_W, nl2_b.reshape(NB, 1, D), owg)
    acc01 = _main01(x, z1, uu1, *wargs, out_b.reshape(1, D))
    acc012 = _main2(acc01, z2, uu2, *wargs)
    out = _main3(acc012, sf3, sf3, su3, su3, inv, *wargs)
    return out
